# Initial kernel scaffold; baseline (speedup 1.0000x reference)
import jax
import jax.numpy as jnp
from jax import lax
from jax.experimental import pallas as pl
from jax.experimental.pallas import tpu as pltpu
from jax.experimental.pallas import tpu_sc as plsc

NC = 2
NS = 16
NW = NC * NS

CHUNK = 128
K = 4
GROUP = CHUNK * K


def _embed_body(idx_hbm, table_hbm, out_hbm, idx_v, rows_v, gsem):
    nchunks = idx_hbm.shape[0] // NW
    ngroups = nchunks // K
    wid = lax.axis_index("s") * NC + lax.axis_index("c")
    cbase = wid * nchunks
    rbase = cbase * CHUNK

    pltpu.sync_copy(idx_hbm.at[pl.ds(cbase, nchunks)], idx_v)

    def group(g, _):
        copies = []
        for b in range(K):
            c = g * K + b
            copies.append(
                pltpu.async_copy(
                    table_hbm.at[idx_v.at[c]],
                    rows_v.at[pl.ds(b * CHUNK, CHUNK)],
                    gsem,
                )
            )
        for cp in copies:
            cp.wait()
        pltpu.sync_copy(rows_v, out_hbm.at[pl.ds(rbase + g * GROUP, GROUP)])

    pl.loop(0, ngroups)(group)


def kernel(x, table):
    B, H = x.shape
    V, D = table.shape
    n = B * H
    idx2d = x.reshape(n // CHUNK, CHUNK)

    run = pl.kernel(
        _embed_body,
        out_type=jax.ShapeDtypeStruct((n, D), table.dtype),
        mesh=plsc.VectorSubcoreMesh(core_axis_name="c", subcore_axis_name="s"),
        scratch_types=[
            pltpu.VMEM((n // CHUNK // NW, CHUNK), jnp.int32),
            pltpu.VMEM((GROUP, D), jnp.float32),
            pltpu.SemaphoreType.DMA,
        ],
    )
    out = run(idx2d, table)
    return out.reshape(B, H, D)


# speedup vs baseline: 3.3930x; 3.3930x over previous
import jax
import jax.numpy as jnp
from jax import lax
from jax.experimental import pallas as pl
from jax.experimental.pallas import tpu as pltpu
from jax.experimental.pallas import tpu_sc as plsc

NC = 2
NS = 16
NW = NC * NS

CHUNK = 128
K = 4
GROUP = CHUNK * K


def _embed_body(idx_hbm, table_hbm, out_hbm, idx_v, rows_v, gsem):
    nchunks = idx_hbm.shape[0] // NW
    ngroups = nchunks // K
    wid = lax.axis_index("s") * NC + lax.axis_index("c")
    cbase = wid * nchunks
    rbase = cbase * CHUNK

    pltpu.sync_copy(idx_hbm.at[pl.ds(cbase, nchunks)], idx_v)

    def group(g):
        copies = []
        for b in range(K):
            c = g * K + b
            copies.append(
                pltpu.async_copy(
                    table_hbm.at[idx_v.at[c]],
                    rows_v.at[pl.ds(b * CHUNK, CHUNK)],
                    gsem,
                )
            )
        for cp in copies:
            cp.wait()
        pltpu.sync_copy(rows_v, out_hbm.at[pl.ds(rbase + g * GROUP, GROUP)])

    pl.loop(0, ngroups)(group)


def kernel(x, table):
    B, H = x.shape
    V, D = table.shape
    n = B * H
    idx2d = x.reshape(n // CHUNK, CHUNK)

    run = pl.kernel(
        _embed_body,
        out_type=jax.ShapeDtypeStruct((n, D), table.dtype),
        mesh=plsc.VectorSubcoreMesh(core_axis_name="c", subcore_axis_name="s"),
        scratch_types=[
            pltpu.VMEM((n // CHUNK // NW, CHUNK), jnp.int32),
            pltpu.VMEM((GROUP, D), jnp.float32),
            pltpu.SemaphoreType.DMA,
        ],
    )
    out = run(idx2d, table)
    return out.reshape(B, H, D)


# speedup vs baseline: 3.4512x; 1.0171x over previous
import jax
import jax.numpy as jnp
from jax import lax
from jax.experimental import pallas as pl
from jax.experimental.pallas import tpu as pltpu
from jax.experimental.pallas import tpu_sc as plsc

NC = 2
NS = 16
NW = NC * NS

CHUNK = 128
K = 2
GROUP = CHUNK * K
NBUF = 3


def _embed_body(idx_hbm, table_hbm, out_hbm, idx_v, rows_v, gsem, wsem):
    nchunks = idx_hbm.shape[0] // NW
    ngroups = nchunks // K
    wid = lax.axis_index("s") * NC + lax.axis_index("c")
    cbase = wid * nchunks
    rbase = cbase * CHUNK

    pltpu.sync_copy(idx_hbm.at[pl.ds(cbase, nchunks)], idx_v)

    def g_copy(g, b, k):
        return (table_hbm.at[idx_v.at[g * K + k]],
                rows_v.at[pl.ds(b * GROUP + k * CHUNK, CHUNK)],
                gsem.at[b])

    def w_copy(g, b):
        return (rows_v.at[pl.ds(b * GROUP, GROUP)],
                out_hbm.at[pl.ds(rbase + g * GROUP, GROUP)],
                wsem.at[b])

    def fire_g(g, b):
        for k in range(K):
            pltpu.async_copy(*g_copy(g, b, k))

    def wait_g(g, b):
        for k in range(K):
            pltpu.make_async_copy(*g_copy(g, b, k)).wait()

    def fire_w(g, b):
        pltpu.async_copy(*w_copy(g, b))

    def wait_w(g, b):
        pltpu.make_async_copy(*w_copy(g, b)).wait()

    fire_g(0, 0)
    fire_g(1, 1)
    wait_g(0, 0)
    fire_w(0, 0)
    fire_g(2, 2)

    def body(g):
        b = g % NBUF
        wait_g(g, b)
        fire_w(g, b)
        b2 = (g + 2) % NBUF
        wait_w(g - 1, b2)
        fire_g(g + 2, b2)

    pl.loop(1, ngroups - 2)(body)

    for g in (ngroups - 2, ngroups - 1):
        b = g % NBUF
        wait_g(g, b)
        fire_w(g, b)
    for g in (ngroups - 3, ngroups - 2, ngroups - 1):
        wait_w(g, g % NBUF)


def kernel(x, table):
    B, H = x.shape
    V, D = table.shape
    n = B * H
    idx2d = x.reshape(n // CHUNK, CHUNK)

    run = pl.kernel(
        _embed_body,
        out_type=jax.ShapeDtypeStruct((n, D), table.dtype),
        mesh=plsc.VectorSubcoreMesh(core_axis_name="c", subcore_axis_name="s"),
        scratch_types=[
            pltpu.VMEM((n // CHUNK // NW, CHUNK), jnp.int32),
            pltpu.VMEM((NBUF * GROUP, D), jnp.float32),
            pltpu.SemaphoreType.DMA((NBUF,)),
            pltpu.SemaphoreType.DMA((NBUF,)),
        ],
    )
    out = run(idx2d, table)
    return out.reshape(B, H, D)


# speedup vs baseline: 3.4531x; 1.0006x over previous
import jax
import jax.numpy as jnp
from jax import lax
from jax.experimental import pallas as pl
from jax.experimental.pallas import tpu as pltpu
from jax.experimental.pallas import tpu_sc as plsc

NC = 2
NS = 16
NW = NC * NS

CHUNK = 128
K = 2
GROUP = CHUNK * K
NBUF = 3


def _embed_body(idx_hbm, table_hbm, out_hbm, idx_v, rows_v, gsem, wsem):
    nchunks = idx_hbm.shape[0] // NW
    ngroups = nchunks // K
    wid = lax.axis_index("s") * NC + lax.axis_index("c")
    cbase = wid * nchunks
    rbase = cbase * CHUNK

    pltpu.sync_copy(idx_hbm.at[pl.ds(cbase, nchunks)], idx_v)

    def g_copy(g, b, k):
        return (table_hbm.at[idx_v.at[g * K + k]],
                rows_v.at[pl.ds(b * GROUP + k * CHUNK, CHUNK)],
                gsem.at[b])

    def w_copy(g, b):
        return (rows_v.at[pl.ds(b * GROUP, GROUP)],
                out_hbm.at[pl.ds(rbase + g * GROUP, GROUP)],
                wsem.at[b])

    def fire_g(g, b):
        for k in range(K):
            pltpu.async_copy(*g_copy(g, b, k))

    def wait_g(g, b):
        for k in range(K):
            pltpu.make_async_copy(*g_copy(g, b, k)).wait()

    def fire_w(g, b):
        pltpu.async_copy(*w_copy(g, b))

    def wait_w(g, b):
        pltpu.make_async_copy(*w_copy(g, b)).wait()

    fire_g(0, 0)
    fire_g(1, 1)
    wait_g(0, 0)
    fire_w(0, 0)
    fire_g(2, 2)

    def body(g):
        b = g % NBUF
        wait_g(g, b)
        fire_w(g, b)
        b2 = (g + 2) % NBUF
        wait_w(g - 1, b2)
        fire_g(g + 2, b2)

    pl.loop(1, ngroups - 2)(body)

    for g in (ngroups - 2, ngroups - 1):
        b = g % NBUF
        wait_g(g, b)
        fire_w(g, b)
    for g in (ngroups - 3, ngroups - 2, ngroups - 1):
        wait_w(g, g % NBUF)


def kernel(x, table):
    B, H = x.shape
    V, D = table.shape
    n = B * H
    idx2d = x.reshape(n // CHUNK, CHUNK)

    run = pl.kernel(
        _embed_body,
        out_type=jax.ShapeDtypeStruct((n, D), table.dtype),
        mesh=plsc.VectorSubcoreMesh(core_axis_name="c", subcore_axis_name="s"),
        scratch_types=[
            pltpu.VMEM((n // CHUNK // NW, CHUNK), jnp.int32),
            pltpu.VMEM((NBUF * GROUP, D), jnp.float32),
            pltpu.SemaphoreType.DMA((NBUF,)),
            pltpu.SemaphoreType.DMA((NBUF,)),
        ],
        compiler_params=pltpu.CompilerParams(use_tc_tiling_on_sc=True),
    )
    out = run(idx2d, table)
    return out.reshape(B, H, D)


# speedup vs baseline: 11.9537x; 3.4617x over previous
import jax
import jax.numpy as jnp
from jax import lax
from jax.experimental import pallas as pl
from jax.experimental.pallas import tpu as pltpu
from jax.experimental.pallas import tpu_sc as plsc

NC = 2
NS = 16
NW = NC * NS

CHUNK = 128
K = 2
GROUP = CHUNK * K
NBUF = 3


def _embed_body(idx_hbm, table_hbm, out_hbm, idx_v, rows_v, gsem, wsem):
    nchunks = idx_hbm.shape[0] // NW
    ngroups = nchunks // K
    wid = lax.axis_index("s") * NC + lax.axis_index("c")
    cbase = wid * nchunks
    rbase = cbase * CHUNK

    pltpu.sync_copy(idx_hbm.at[pl.ds(cbase, nchunks)], idx_v)

    def g_copy(g, b, k):
        return (table_hbm.at[idx_v.at[g * K + k]],
                rows_v.at[pl.ds(b * GROUP + k * CHUNK, CHUNK)],
                gsem.at[b])

    def w_copy(g, b):
        return (rows_v.at[pl.ds(b * GROUP, GROUP)],
                out_hbm.at[pl.ds(rbase + g * GROUP, GROUP)],
                wsem.at[b])

    def fire_g(g, b):
        for k in range(K):
            pltpu.async_copy(*g_copy(g, b, k))

    def wait_g(g, b):
        for k in range(K):
            pltpu.make_async_copy(*g_copy(g, b, k)).wait()

    def fire_w(g, b):
        pltpu.async_copy(*w_copy(g, b))

    def wait_w(g, b):
        pltpu.make_async_copy(*w_copy(g, b)).wait()

    fire_g(0, 0)
    fire_g(1, 1)
    wait_g(0, 0)
    fire_w(0, 0)
    fire_g(2, 2)

    def body(g):
        b = g % NBUF
        wait_g(g, b)
        fire_w(g, b)
        b2 = (g + 2) % NBUF
        wait_w(g - 1, b2)
        fire_g(g + 2, b2)

    pl.loop(1, ngroups - 2)(body)

    for g in (ngroups - 2, ngroups - 1):
        b = g % NBUF
        wait_g(g, b)
        fire_w(g, b)
    for g in (ngroups - 3, ngroups - 2, ngroups - 1):
        wait_w(g, g % NBUF)


def kernel(x, table):
    B, H = x.shape
    V, D = table.shape
    n = B * H
    idx2d = x.T.reshape(n // CHUNK, CHUNK)

    run = pl.kernel(
        _embed_body,
        out_type=jax.ShapeDtypeStruct((n, D), table.dtype),
        mesh=plsc.VectorSubcoreMesh(core_axis_name="c", subcore_axis_name="s"),
        scratch_types=[
            pltpu.VMEM((n // CHUNK // NW, CHUNK), jnp.int32),
            pltpu.VMEM((NBUF * GROUP, D), jnp.float32),
            pltpu.SemaphoreType.DMA((NBUF,)),
            pltpu.SemaphoreType.DMA((NBUF,)),
        ],
        compiler_params=pltpu.CompilerParams(use_tc_tiling_on_sc=True),
    )
    out = run(idx2d, table)
    return out.reshape(H, B, D).transpose(1, 0, 2)


# speedup vs baseline: 12.0096x; 1.0047x over previous
import jax
import jax.numpy as jnp
from jax import lax
from jax.experimental import pallas as pl
from jax.experimental.pallas import tpu as pltpu
from jax.experimental.pallas import tpu_sc as plsc

NC = 2
NS = 16
NW = NC * NS

CHUNK = 128
K = 1
GROUP = CHUNK * K
NBUF = 6


def _embed_body(idx_hbm, table_hbm, out_hbm, idx_v, rows_v, gsem, wsem):
    nchunks = idx_hbm.shape[0] // NW
    ngroups = nchunks // K
    wid = lax.axis_index("s") * NC + lax.axis_index("c")
    cbase = wid * nchunks
    rbase = cbase * CHUNK

    pltpu.sync_copy(idx_hbm.at[pl.ds(cbase, nchunks)], idx_v)

    def g_copy(g, b, k):
        return (table_hbm.at[idx_v.at[g * K + k]],
                rows_v.at[pl.ds(b * GROUP + k * CHUNK, CHUNK)],
                gsem.at[b])

    def w_copy(g, b):
        return (rows_v.at[pl.ds(b * GROUP, GROUP)],
                out_hbm.at[pl.ds(rbase + g * GROUP, GROUP)],
                wsem.at[b])

    def fire_g(g, b):
        for k in range(K):
            pltpu.async_copy(*g_copy(g, b, k))

    def wait_g(g, b):
        for k in range(K):
            pltpu.make_async_copy(*g_copy(g, b, k)).wait()

    def fire_w(g, b):
        pltpu.async_copy(*w_copy(g, b))

    def wait_w(g, b):
        pltpu.make_async_copy(*w_copy(g, b)).wait()

    for j in range(NBUF - 1):
        fire_g(j, j)
    wait_g(0, 0)
    fire_w(0, 0)
    fire_g(NBUF - 1, NBUF - 1)

    def body(g):
        b = g % NBUF
        wait_g(g, b)
        fire_w(g, b)
        b2 = (g + NBUF - 1) % NBUF
        wait_w(g - 1, b2)
        fire_g(g + NBUF - 1, b2)

    pl.loop(1, ngroups - NBUF + 1)(body)

    for g in range(ngroups - NBUF + 1, ngroups):
        b = g % NBUF
        wait_g(g, b)
        fire_w(g, b)
    for g in range(ngroups - NBUF, ngroups):
        wait_w(g, g % NBUF)


def kernel(x, table):
    B, H = x.shape
    V, D = table.shape
    n = B * H
    idx2d = x.T.reshape(n // CHUNK, CHUNK)

    run = pl.kernel(
        _embed_body,
        out_type=jax.ShapeDtypeStruct((n, D), table.dtype),
        mesh=plsc.VectorSubcoreMesh(core_axis_name="c", subcore_axis_name="s"),
        scratch_types=[
            pltpu.VMEM((n // CHUNK // NW, CHUNK), jnp.int32),
            pltpu.VMEM((NBUF * GROUP, D), jnp.float32),
            pltpu.SemaphoreType.DMA((NBUF,)),
            pltpu.SemaphoreType.DMA((NBUF,)),
        ],
        compiler_params=pltpu.CompilerParams(use_tc_tiling_on_sc=True),
    )
    out = run(idx2d, table)
    return out.reshape(H, B, D).transpose(1, 0, 2)
